# baseline (speedup 1.0000x reference)
import functools

import jax
import jax.numpy as jnp
from jax import lax
from jax.experimental import pallas as pl
from jax.experimental.pallas import tpu as pltpu
from jax.experimental.pallas import tpu_sc as plsc

N = 10000
E = 320000
H = 128
P = 16384

NC = 2
NS = 16
NW = NC * NS
EW = 10240
E_PAD = NW * EW
ACC_ROWS = 10112
RPT = ACC_ROWS // NS

CHD = 128
GD = EW // CHD
CH = 64
G = EW // CH

_mesh = plsc.VectorSubcoreMesh(core_axis_name="c", subcore_axis_name="s")
_HIGH = jax.lax.Precision.HIGHEST
_SC_PARAMS = pltpu.CompilerParams(use_tc_tiling_on_sc=False)


def _worker_id():
    return lax.axis_index("s") * NC + lax.axis_index("c")


@functools.partial(
    pl.kernel,
    out_type=jax.ShapeDtypeStruct((NC, ACC_ROWS, 16), jnp.float32),
    mesh=_mesh,
    compiler_params=_SC_PARAMS,
    scratch_types=[
        pltpu.VMEM((GD, CHD), jnp.int32),
        pltpu.VMEM((CHD, 16), jnp.float32),
        pltpu.VMEM_SHARED((ACC_ROWS, 16), jnp.float32),
        pltpu.SemaphoreType.DMA,
    ],
)
def _deg(dstw, ones, zeros, out, dst_v, ones_v, acc, sem):
    c = lax.axis_index("c")
    s = lax.axis_index("s")
    w = _worker_id()
    pltpu.sync_copy(dstw.at[w], dst_v)
    pltpu.sync_copy(ones, ones_v)
    pltpu.sync_copy(zeros.at[pl.ds(s * RPT, RPT)], acc.at[pl.ds(s * RPT, RPT)])
    plsc.subcore_barrier()

    def fire(g, _):
        pltpu.async_copy(ones_v, acc.at[dst_v.at[g]], sem, add=True)
        return 0

    def drain(g, _):
        pltpu.make_async_copy(ones_v, acc.at[dst_v.at[g]], sem).wait()
        return 0

    lax.fori_loop(0, GD, fire, 0)
    lax.fori_loop(0, GD, drain, 0)
    plsc.subcore_barrier()
    pltpu.sync_copy(acc.at[pl.ds(s * RPT, RPT)], out.at[c].at[pl.ds(s * RPT, RPT)])


@functools.partial(
    pl.kernel,
    out_type=jax.ShapeDtypeStruct((NC, ACC_ROWS, H), jnp.float32),
    mesh=_mesh,
    compiler_params=_SC_PARAMS,
    scratch_types=[
        pltpu.VMEM((G, CH), jnp.int32),
        pltpu.VMEM((G, CH), jnp.int32),
        pltpu.VMEM((2, CH, H), jnp.float32),
        pltpu.VMEM_SHARED((ACC_ROWS, H), jnp.float32),
    ]
    + [pltpu.SemaphoreType.DMA] * 4,
)
def _agg(hp, srcw, dstw, zeros, out, src_v, dst_v, bufs, acc, g0, g1, s0, s1):
    c = lax.axis_index("c")
    s = lax.axis_index("s")
    w = _worker_id()
    gsem = (g0, g1)
    ssem = (s0, s1)
    pltpu.sync_copy(srcw.at[w], src_v)
    pltpu.sync_copy(dstw.at[w], dst_v)
    pltpu.sync_copy(zeros.at[pl.ds(s * RPT, RPT)], acc.at[pl.ds(s * RPT, RPT)])
    plsc.subcore_barrier()

    pltpu.async_copy(hp.at[src_v.at[0]], bufs.at[0], g0)
    pltpu.async_copy(hp.at[src_v.at[1]], bufs.at[1], g1)

    def rnd(i, _):
        for k in range(2):
            g = 2 * i + k
            pltpu.make_async_copy(hp.at[src_v.at[g]], bufs.at[k], gsem[k]).wait()
            pltpu.async_copy(bufs.at[k], acc.at[dst_v.at[g]], ssem[k], add=True)
        for k in range(2):
            g = 2 * i + k
            pltpu.make_async_copy(bufs.at[k], acc.at[dst_v.at[g]], ssem[k]).wait()
            pltpu.async_copy(hp.at[src_v.at[g + 2]], bufs.at[k], gsem[k])
        return 0

    lax.fori_loop(0, G // 2 - 1, rnd, 0)
    gl = G - 2
    for k in range(2):
        pltpu.make_async_copy(hp.at[src_v.at[gl + k]], bufs.at[k], gsem[k]).wait()
        pltpu.async_copy(bufs.at[k], acc.at[dst_v.at[gl + k]], ssem[k], add=True)
    for k in range(2):
        pltpu.make_async_copy(bufs.at[k], acc.at[dst_v.at[gl + k]], ssem[k]).wait()
    plsc.subcore_barrier()
    pltpu.sync_copy(acc.at[pl.ds(s * RPT, RPT)], out.at[c].at[pl.ds(s * RPT, RPT)])


CHP = 128
PG = (2 * P) // NW // CHP


@functools.partial(
    pl.kernel,
    out_type=jax.ShapeDtypeStruct((2 * P, H), jnp.float32),
    mesh=_mesh,
    scratch_types=[
        pltpu.VMEM((PG, CHP), jnp.int32),
        pltpu.VMEM((CHP, H), jnp.float32),
        pltpu.VMEM((CHP, H), jnp.float32),
        pltpu.SemaphoreType.DMA,
        pltpu.SemaphoreType.DMA,
    ],
)
def _pairs(h3, idxw, out, idx_v, buf_a, buf_b, sem_a, sem_b):
    w = _worker_id()
    base = w * (PG * CHP)
    pltpu.sync_copy(idxw.at[w], idx_v)
    bufs = (buf_a, buf_b)
    sems = (sem_a, sem_b)
    pltpu.async_copy(h3.at[idx_v.at[0]], buf_a, sem_a)
    pltpu.async_copy(h3.at[idx_v.at[1]], buf_b, sem_b)
    for g in range(PG):
        b, sm = bufs[g % 2], sems[g % 2]
        pltpu.make_async_copy(h3.at[idx_v.at[g]], b, sm).wait()
        pltpu.sync_copy(b, out.at[pl.ds(base + g * CHP, CHP)])
        if g + 2 < PG:
            pltpu.async_copy(h3.at[idx_v.at[g + 2]], b, sm)


BR = 2000
GRID_N = N // BR


def _sp_body(x_ref, w_ref, dp_ref, dis_ref, dinv_ref, hw_ref, hp_ref):
    hw = jnp.dot(x_ref[...], w_ref[...], precision=_HIGH,
                 preferred_element_type=jnp.float32)
    deg = dp_ref[0, :, 0:1] + dp_ref[1, :, 0:1] + 1.0
    dis = lax.rsqrt(deg)
    dis_ref[...] = dis
    dinv_ref[...] = 1.0 / deg
    hw_ref[...] = hw
    hp_ref[...] = hw * dis


def _mid_body(sp_ref, hw_ref, dis_ref, dinv_ref, b_ref, w_ref,
              hwo_ref, hp_ref):
    dis = dis_ref[...]
    ss = sp_ref[0] + sp_ref[1]
    t = dis * ss + dinv_ref[...] * hw_ref[...] + b_ref[...]
    t = jnp.maximum(t, 0.0)
    hw2 = jnp.dot(t, w_ref[...], precision=_HIGH,
                  preferred_element_type=jnp.float32)
    hwo_ref[...] = hw2
    hp_ref[...] = hw2 * dis


def _fin_body(sp_ref, hw_ref, dis_ref, dinv_ref, b_ref, h3_ref):
    ss = sp_ref[0] + sp_ref[1]
    h3_ref[...] = (dis_ref[...] * ss + dinv_ref[...] * hw_ref[...] + b_ref[...])


BP = 512
GRID_P = P // BP


def _pred_body(gs_ref, gd_ref, wa_ref, wb_ref, bp1_ref, wp2_ref, bp2_ref, out_ref):
    e = (jnp.dot(gs_ref[...], wa_ref[...], precision=_HIGH,
                 preferred_element_type=jnp.float32)
         + jnp.dot(gd_ref[...], wb_ref[...], precision=_HIGH,
                   preferred_element_type=jnp.float32)
         + bp1_ref[...])
    e = jnp.maximum(e, 0.0)
    z = jnp.sum(e * wp2_ref[...], axis=1, keepdims=True) + bp2_ref[...]
    out_ref[...] = 1.0 / (1.0 + jnp.exp(-z))


def _row_spec(shape):
    return pl.BlockSpec(shape, lambda i: (i, 0))


def _full_spec(shape):
    return pl.BlockSpec(shape, lambda i: (0, 0))


_s_spec = pl.BlockSpec((NC, BR, H), lambda i: (0, i, 0))
_d16_spec = pl.BlockSpec((NC, BR, 16), lambda i: (0, i, 0))

_tc_sp = pl.pallas_call(
    _sp_body,
    grid=(GRID_N,),
    in_specs=[_row_spec((BR, H)), _full_spec((H, H)), _d16_spec],
    out_specs=[_row_spec((BR, 1)), _row_spec((BR, 1)),
               _row_spec((BR, H)), _row_spec((BR, H))],
    out_shape=[jax.ShapeDtypeStruct((N, 1), jnp.float32),
               jax.ShapeDtypeStruct((N, 1), jnp.float32),
               jax.ShapeDtypeStruct((N, H), jnp.float32),
               jax.ShapeDtypeStruct((N, H), jnp.float32)],
)

_tc_mid = pl.pallas_call(
    _mid_body,
    grid=(GRID_N,),
    in_specs=[_s_spec, _row_spec((BR, H)), _row_spec((BR, 1)),
              _row_spec((BR, 1)), _full_spec((1, H)), _full_spec((H, H))],
    out_specs=[_row_spec((BR, H)), _row_spec((BR, H))],
    out_shape=[jax.ShapeDtypeStruct((N, H), jnp.float32),
               jax.ShapeDtypeStruct((N, H), jnp.float32)],
)

_tc_fin = pl.pallas_call(
    _fin_body,
    grid=(GRID_N,),
    in_specs=[_s_spec, _row_spec((BR, H)), _row_spec((BR, 1)),
              _row_spec((BR, 1)), _full_spec((1, H))],
    out_specs=_row_spec((BR, H)),
    out_shape=jax.ShapeDtypeStruct((N, H), jnp.float32),
)

_tc_pred = pl.pallas_call(
    _pred_body,
    grid=(GRID_P,),
    in_specs=[_row_spec((BP, H)), _row_spec((BP, H)), _full_spec((H, H)),
              _full_spec((H, H)), _full_spec((1, H)), _full_spec((1, H)),
              _full_spec((1, 1))],
    out_specs=_row_spec((BP, 1)),
    out_shape=jax.ShapeDtypeStruct((P, 1), jnp.float32),
)


def kernel(x, edge_index, edge_pairs, W1, b1, W2, b2, W3, b3, Wp1, bp1, Wp2, bp2):
    src = edge_index[0]
    dst = edge_index[1]
    npad = E_PAD - E
    pad_i = jnp.arange(npad, dtype=jnp.int32)
    src_flat = jnp.concatenate([src, (pad_i * 997) % N])
    dst_flat = jnp.concatenate([dst, N + (pad_i % 16)])
    srcw = src_flat.reshape(NW, G, CH)
    dstw = dst_flat.reshape(NW, G, CH)
    dstw_deg = dst_flat.reshape(NW, GD, CHD)
    idxw = jnp.concatenate([edge_pairs[0], edge_pairs[1]]).reshape(NW, PG, CHP)

    zeros_h = jnp.zeros((ACC_ROWS, H), jnp.float32)
    zeros16 = jnp.zeros((ACC_ROWS, 16), jnp.float32)
    ones16 = jnp.ones((CHD, 16), jnp.float32)

    deg_parts = _deg(dstw_deg, ones16, zeros16)
    dis, dinv, hw1, hp1 = _tc_sp(x, W1, deg_parts)

    s1 = _agg(hp1, srcw, dstw, zeros_h)
    hw2, hp2 = _tc_mid(s1, hw1, dis, dinv, b1.reshape(1, H), W2)
    s2 = _agg(hp2, srcw, dstw, zeros_h)
    hw3, hp3 = _tc_mid(s2, hw2, dis, dinv, b2.reshape(1, H), W3)
    s3 = _agg(hp3, srcw, dstw, zeros_h)
    h3 = _tc_fin(s3, hw3, dis, dinv, b3.reshape(1, H))

    g = _pairs(h3, idxw)
    pred = _tc_pred(g[:P], g[P:], Wp1[:H], Wp1[H:], bp1.reshape(1, H),
                    Wp2.reshape(1, H), bp2.reshape(1, 1))
    return pred.reshape(P)


# speedup vs baseline: 1.0577x; 1.0577x over previous
import functools

import jax
import jax.numpy as jnp
from jax import lax
from jax.experimental import pallas as pl
from jax.experimental.pallas import tpu as pltpu
from jax.experimental.pallas import tpu_sc as plsc

N = 10000
E = 320000
H = 128
P = 16384

NC = 2
NS = 16
NW = NC * NS
CH = 128
G = 80
EW = G * CH
E_PAD = NW * EW
ACC_ROWS = 10112
RPT = ACC_ROWS // NS

_mesh = plsc.VectorSubcoreMesh(core_axis_name="c", subcore_axis_name="s")
_HIGH = jax.lax.Precision.HIGHEST
_SC_PARAMS = pltpu.CompilerParams(use_tc_tiling_on_sc=False)
HH = H // 2


def _worker_id():
    return lax.axis_index("s") * NC + lax.axis_index("c")


@functools.partial(
    pl.kernel,
    out_type=jax.ShapeDtypeStruct((NC, ACC_ROWS, 16), jnp.float32),
    mesh=_mesh,
    compiler_params=_SC_PARAMS,
    scratch_types=[
        pltpu.VMEM((G, CH), jnp.int32),
        pltpu.VMEM((CH, 16), jnp.float32),
        pltpu.VMEM_SHARED((ACC_ROWS, 16), jnp.float32),
        pltpu.SemaphoreType.DMA,
        pltpu.SemaphoreType.DMA,
        pltpu.SemaphoreType.DMA,
    ],
)
def _deg(dstw, ones, zeros, out, dst_v, ones_v, acc, sem, p0, p1):
    c = lax.axis_index("c")
    s = lax.axis_index("s")
    w = _worker_id()
    pltpu.async_copy(dstw.at[w], dst_v, p0)
    pltpu.async_copy(ones, ones_v, p1)
    pltpu.async_copy(zeros.at[pl.ds(s * RPT, RPT)], acc.at[pl.ds(s * RPT, RPT)], sem)
    pltpu.make_async_copy(dstw.at[w], dst_v, p0).wait()
    pltpu.make_async_copy(ones, ones_v, p1).wait()
    pltpu.make_async_copy(zeros.at[pl.ds(s * RPT, RPT)],
                          acc.at[pl.ds(s * RPT, RPT)], sem).wait()
    plsc.subcore_barrier()

    def fire(g, _):
        pltpu.async_copy(ones_v, acc.at[dst_v.at[g]], sem, add=True)
        return 0

    def drain(g, _):
        pltpu.make_async_copy(ones_v, acc.at[dst_v.at[g]], sem).wait()
        return 0

    lax.fori_loop(0, G, fire, 0)
    lax.fori_loop(0, G, drain, 0)
    plsc.subcore_barrier()
    pltpu.sync_copy(acc.at[pl.ds(s * RPT, RPT)], out.at[c].at[pl.ds(s * RPT, RPT)])


@functools.partial(
    pl.kernel,
    out_type=[jax.ShapeDtypeStruct((NC, ACC_ROWS, HH), jnp.float32)] * 2,
    mesh=_mesh,
    compiler_params=_SC_PARAMS,
    scratch_types=[
        pltpu.VMEM((G, CH), jnp.int32),
        pltpu.VMEM((G, CH), jnp.int32),
        pltpu.VMEM((2, CH, HH), jnp.float32),
        pltpu.VMEM_SHARED((ACC_ROWS, HH), jnp.float32),
        pltpu.SemaphoreType.DMA,
        pltpu.SemaphoreType.DMA,
        pltpu.SemaphoreType.DMA,
    ],
)
def _agg(hp_lo, hp_hi, srcw, dstw, zeros, out_lo, out_hi,
         src_v, dst_v, bufs, acc, g0, g1, zsem):
    c = lax.axis_index("c")
    s = lax.axis_index("s")
    w = _worker_id()
    gsem = (g0, g1)
    pltpu.async_copy(srcw.at[w], src_v, g0)
    pltpu.async_copy(dstw.at[w], dst_v, g1)
    pltpu.async_copy(zeros.at[pl.ds(s * RPT, RPT)],
                     acc.at[pl.ds(s * RPT, RPT)], zsem)
    pltpu.make_async_copy(srcw.at[w], src_v, g0).wait()
    pltpu.make_async_copy(dstw.at[w], dst_v, g1).wait()
    pltpu.make_async_copy(zeros.at[pl.ds(s * RPT, RPT)],
                          acc.at[pl.ds(s * RPT, RPT)], zsem).wait()

    for half, (hp, out) in enumerate(((hp_lo, out_lo), (hp_hi, out_hi))):
        if half:
            pltpu.make_async_copy(zeros.at[pl.ds(s * RPT, RPT)],
                                  acc.at[pl.ds(s * RPT, RPT)], zsem).wait()
        plsc.subcore_barrier()
        pltpu.async_copy(hp.at[src_v.at[0]], bufs.at[0], g0)
        pltpu.async_copy(hp.at[src_v.at[1]], bufs.at[1], g1)

        def rnd(i, _):
            for k in range(2):
                g = 2 * i + k
                pltpu.make_async_copy(hp.at[src_v.at[g]], bufs.at[k], gsem[k]).wait()
                pltpu.sync_copy(bufs.at[k], acc.at[dst_v.at[g]], add=True)
                pltpu.async_copy(hp.at[src_v.at[g + 2]], bufs.at[k], gsem[k])
            return 0

        lax.fori_loop(0, G // 2 - 1, rnd, 0)
        gl = G - 2
        for k in range(2):
            pltpu.make_async_copy(hp.at[src_v.at[gl + k]], bufs.at[k], gsem[k]).wait()
            pltpu.sync_copy(bufs.at[k], acc.at[dst_v.at[gl + k]], add=True)
        plsc.subcore_barrier()
        pltpu.sync_copy(acc.at[pl.ds(s * RPT, RPT)],
                        out.at[c].at[pl.ds(s * RPT, RPT)])
        if not half:
            pltpu.async_copy(zeros.at[pl.ds(s * RPT, RPT)],
                             acc.at[pl.ds(s * RPT, RPT)], zsem)


PG = (2 * P) // NW // CH


@functools.partial(
    pl.kernel,
    out_type=jax.ShapeDtypeStruct((2 * P, H), jnp.float32),
    mesh=_mesh,
    scratch_types=[
        pltpu.VMEM((PG, CH), jnp.int32),
        pltpu.VMEM((CH, H), jnp.float32),
        pltpu.VMEM((CH, H), jnp.float32),
        pltpu.SemaphoreType.DMA,
        pltpu.SemaphoreType.DMA,
    ],
)
def _pairs(h3, idxw, out, idx_v, buf_a, buf_b, sem_a, sem_b):
    w = _worker_id()
    base = w * (PG * CH)
    pltpu.sync_copy(idxw.at[w], idx_v)
    bufs = (buf_a, buf_b)
    sems = (sem_a, sem_b)
    pltpu.async_copy(h3.at[idx_v.at[0]], buf_a, sem_a)
    pltpu.async_copy(h3.at[idx_v.at[1]], buf_b, sem_b)
    for g in range(PG):
        b, sm = bufs[g % 2], sems[g % 2]
        pltpu.make_async_copy(h3.at[idx_v.at[g]], b, sm).wait()
        pltpu.sync_copy(b, out.at[pl.ds(base + g * CH, CH)])
        if g + 2 < PG:
            pltpu.async_copy(h3.at[idx_v.at[g + 2]], b, sm)


BR = 2000
GRID_N = N // BR


def _mm_body(x_ref, w_ref, hw_ref):
    hw_ref[...] = jnp.dot(x_ref[...], w_ref[...], precision=_HIGH,
                          preferred_element_type=jnp.float32)


def _sp_body(dp_ref, hw_ref, dis_ref, dinv_ref, hplo_ref, hphi_ref):
    deg = dp_ref[0, :, 0:1] + dp_ref[1, :, 0:1] + 1.0
    dis = lax.rsqrt(deg)
    dis_ref[...] = dis
    dinv_ref[...] = 1.0 / deg
    hp = hw_ref[...] * dis
    hplo_ref[...] = hp[:, :HH]
    hphi_ref[...] = hp[:, HH:]


def _mid_body(slo_ref, shi_ref, hw_ref, dis_ref, dinv_ref, b_ref, w_ref,
              hwo_ref, hplo_ref, hphi_ref):
    dis = dis_ref[...]
    ss = jnp.concatenate([slo_ref[0] + slo_ref[1], shi_ref[0] + shi_ref[1]],
                         axis=1)
    t = dis * ss + dinv_ref[...] * hw_ref[...] + b_ref[...]
    t = jnp.maximum(t, 0.0)
    hw2 = jnp.dot(t, w_ref[...], precision=_HIGH,
                  preferred_element_type=jnp.float32)
    hwo_ref[...] = hw2
    hp = hw2 * dis
    hplo_ref[...] = hp[:, :HH]
    hphi_ref[...] = hp[:, HH:]


def _fin_body(slo_ref, shi_ref, hw_ref, dis_ref, dinv_ref, b_ref, h3_ref):
    ss = jnp.concatenate([slo_ref[0] + slo_ref[1], shi_ref[0] + shi_ref[1]],
                         axis=1)
    h3_ref[...] = (dis_ref[...] * ss + dinv_ref[...] * hw_ref[...] + b_ref[...])


BP = 512
GRID_P = P // BP


def _pred_body(gs_ref, gd_ref, wa_ref, wb_ref, bp1_ref, wp2_ref, bp2_ref, out_ref):
    e = (jnp.dot(gs_ref[...], wa_ref[...], precision=_HIGH,
                 preferred_element_type=jnp.float32)
         + jnp.dot(gd_ref[...], wb_ref[...], precision=_HIGH,
                   preferred_element_type=jnp.float32)
         + bp1_ref[...])
    e = jnp.maximum(e, 0.0)
    z = jnp.sum(e * wp2_ref[...], axis=1, keepdims=True) + bp2_ref[...]
    out_ref[...] = 1.0 / (1.0 + jnp.exp(-z))


def _row_spec(shape):
    return pl.BlockSpec(shape, lambda i: (i, 0))


def _full_spec(shape):
    return pl.BlockSpec(shape, lambda i: (0, 0))


_sh_spec = pl.BlockSpec((NC, BR, HH), lambda i: (0, i, 0))
_d16_spec = pl.BlockSpec((NC, BR, 16), lambda i: (0, i, 0))

_tc_mm = pl.pallas_call(
    _mm_body,
    grid=(GRID_N,),
    in_specs=[_row_spec((BR, H)), _full_spec((H, H))],
    out_specs=_row_spec((BR, H)),
    out_shape=jax.ShapeDtypeStruct((N, H), jnp.float32),
)

_tc_sp = pl.pallas_call(
    _sp_body,
    grid=(GRID_N,),
    in_specs=[_d16_spec, _row_spec((BR, H))],
    out_specs=[_row_spec((BR, 1)), _row_spec((BR, 1)),
               _row_spec((BR, HH)), _row_spec((BR, HH))],
    out_shape=[jax.ShapeDtypeStruct((N, 1), jnp.float32),
               jax.ShapeDtypeStruct((N, 1), jnp.float32),
               jax.ShapeDtypeStruct((N, HH), jnp.float32),
               jax.ShapeDtypeStruct((N, HH), jnp.float32)],
)

_tc_mid = pl.pallas_call(
    _mid_body,
    grid=(GRID_N,),
    in_specs=[_sh_spec, _sh_spec, _row_spec((BR, H)), _row_spec((BR, 1)),
              _row_spec((BR, 1)), _full_spec((1, H)), _full_spec((H, H))],
    out_specs=[_row_spec((BR, H)), _row_spec((BR, HH)), _row_spec((BR, HH))],
    out_shape=[jax.ShapeDtypeStruct((N, H), jnp.float32),
               jax.ShapeDtypeStruct((N, HH), jnp.float32),
               jax.ShapeDtypeStruct((N, HH), jnp.float32)],
)

_tc_fin = pl.pallas_call(
    _fin_body,
    grid=(GRID_N,),
    in_specs=[_sh_spec, _sh_spec, _row_spec((BR, H)), _row_spec((BR, 1)),
              _row_spec((BR, 1)), _full_spec((1, H))],
    out_specs=_row_spec((BR, H)),
    out_shape=jax.ShapeDtypeStruct((N, H), jnp.float32),
)

_tc_pred = pl.pallas_call(
    _pred_body,
    grid=(GRID_P,),
    in_specs=[_row_spec((BP, H)), _row_spec((BP, H)), _full_spec((H, H)),
              _full_spec((H, H)), _full_spec((1, H)), _full_spec((1, H)),
              _full_spec((1, 1))],
    out_specs=_row_spec((BP, 1)),
    out_shape=jax.ShapeDtypeStruct((P, 1), jnp.float32),
)


def kernel(x, edge_index, edge_pairs, W1, b1, W2, b2, W3, b3, Wp1, bp1, Wp2, bp2):
    src = edge_index[0]
    dst = edge_index[1]
    npad = E_PAD - E
    pad_i = jnp.arange(npad, dtype=jnp.int32)
    srcw = jnp.concatenate([src, (pad_i * 997) % N]).reshape(NW, G, CH)
    dstw = jnp.concatenate([dst, N + (pad_i % 16)]).reshape(NW, G, CH)
    idxw = jnp.concatenate([edge_pairs[0], edge_pairs[1]]).reshape(NW, PG, CH)

    zeros_h = jnp.zeros((ACC_ROWS, HH), jnp.float32)
    zeros16 = jnp.zeros((ACC_ROWS, 16), jnp.float32)
    ones16 = jnp.ones((CH, 16), jnp.float32)

    deg_parts = _deg(dstw, ones16, zeros16)
    hw1 = _tc_mm(x, W1)
    dis, dinv, hp1l, hp1h = _tc_sp(deg_parts, hw1)

    s1l, s1h = _agg(hp1l, hp1h, srcw, dstw, zeros_h)
    hw2, hp2l, hp2h = _tc_mid(s1l, s1h, hw1, dis, dinv, b1.reshape(1, H), W2)
    s2l, s2h = _agg(hp2l, hp2h, srcw, dstw, zeros_h)
    hw3, hp3l, hp3h = _tc_mid(s2l, s2h, hw2, dis, dinv, b2.reshape(1, H), W3)
    s3l, s3h = _agg(hp3l, hp3h, srcw, dstw, zeros_h)
    h3 = _tc_fin(s3l, s3h, hw3, dis, dinv, b3.reshape(1, H))

    g = _pairs(h3, idxw)
    pred = _tc_pred(g[:P], g[P:], Wp1[:H], Wp1[H:], bp1.reshape(1, H),
                    Wp2.reshape(1, H), bp2.reshape(1, 1))
    return pred.reshape(P)


# speedup vs baseline: 1.1450x; 1.0825x over previous
import functools

import jax
import jax.numpy as jnp
from jax import lax
from jax.experimental import pallas as pl
from jax.experimental.pallas import tpu as pltpu
from jax.experimental.pallas import tpu_sc as plsc

N = 10000
E = 320000
H = 128
P = 16384

NC = 2
NS = 16
NW = NC * NS
CH = 128
G = 80
EW = G * CH
E_PAD = NW * EW
ACC_ROWS = 10112
RPT = ACC_ROWS // NS

_mesh = plsc.VectorSubcoreMesh(core_axis_name="c", subcore_axis_name="s")
_HIGH = jax.lax.Precision.HIGHEST
_SC_PARAMS = pltpu.CompilerParams(use_tc_tiling_on_sc=False)
HH = H // 2


def _worker_id():
    return lax.axis_index("s") * NC + lax.axis_index("c")


@functools.partial(
    pl.kernel,
    out_type=jax.ShapeDtypeStruct((NC, ACC_ROWS, 16), jnp.float32),
    mesh=_mesh,
    compiler_params=_SC_PARAMS,
    scratch_types=[
        pltpu.VMEM((G, CH), jnp.int32),
        pltpu.VMEM((CH, 16), jnp.float32),
        pltpu.VMEM_SHARED((ACC_ROWS, 16), jnp.float32),
        pltpu.SemaphoreType.DMA,
        pltpu.SemaphoreType.DMA,
        pltpu.SemaphoreType.DMA,
    ],
)
def _deg(dstw, ones, zeros, out, dst_v, ones_v, acc, sem, p0, p1):
    c = lax.axis_index("c")
    s = lax.axis_index("s")
    w = _worker_id()
    pltpu.async_copy(dstw.at[w], dst_v, p0)
    pltpu.async_copy(ones, ones_v, p1)
    pltpu.async_copy(zeros.at[pl.ds(s * RPT, RPT)], acc.at[pl.ds(s * RPT, RPT)], sem)
    pltpu.make_async_copy(dstw.at[w], dst_v, p0).wait()
    pltpu.make_async_copy(ones, ones_v, p1).wait()
    pltpu.make_async_copy(zeros.at[pl.ds(s * RPT, RPT)],
                          acc.at[pl.ds(s * RPT, RPT)], sem).wait()
    plsc.subcore_barrier()

    def fire(g, _):
        pltpu.async_copy(ones_v, acc.at[dst_v.at[g]], sem, add=True)
        return 0

    def drain(g, _):
        pltpu.make_async_copy(ones_v, acc.at[dst_v.at[g]], sem).wait()
        return 0

    lax.fori_loop(0, G, fire, 0)
    lax.fori_loop(0, G, drain, 0)
    plsc.subcore_barrier()
    pltpu.sync_copy(acc.at[pl.ds(s * RPT, RPT)], out.at[c].at[pl.ds(s * RPT, RPT)])


K = 5


@functools.partial(
    pl.kernel,
    out_type=[jax.ShapeDtypeStruct((NC, ACC_ROWS, HH), jnp.float32)] * 2,
    mesh=_mesh,
    compiler_params=_SC_PARAMS,
    scratch_types=[
        pltpu.VMEM((G, CH), jnp.int32),
        pltpu.VMEM((G, CH), jnp.int32),
        pltpu.VMEM((K, CH, HH), jnp.float32),
        pltpu.VMEM_SHARED((ACC_ROWS, HH), jnp.float32),
    ]
    + [pltpu.SemaphoreType.DMA] * (2 * K),
)
def _agg(hp_lo, hp_hi, srcw, dstw, zeros, out_lo, out_hi,
         src_v, dst_v, bufs, acc, *sems):
    c = lax.axis_index("c")
    s = lax.axis_index("s")
    w = _worker_id()
    gsem = sems[:K]
    ssem = sems[K:]
    pltpu.async_copy(srcw.at[w], src_v, gsem[0])
    pltpu.async_copy(dstw.at[w], dst_v, gsem[1])
    pltpu.async_copy(zeros.at[pl.ds(s * RPT, RPT)],
                     acc.at[pl.ds(s * RPT, RPT)], ssem[0])
    pltpu.make_async_copy(srcw.at[w], src_v, gsem[0]).wait()
    pltpu.make_async_copy(dstw.at[w], dst_v, gsem[1]).wait()
    pltpu.make_async_copy(zeros.at[pl.ds(s * RPT, RPT)],
                          acc.at[pl.ds(s * RPT, RPT)], ssem[0]).wait()
    R = G // K

    for half, (hp, out) in enumerate(((hp_lo, out_lo), (hp_hi, out_hi))):
        if half:
            pltpu.make_async_copy(zeros.at[pl.ds(s * RPT, RPT)],
                                  acc.at[pl.ds(s * RPT, RPT)], ssem[0]).wait()
        plsc.subcore_barrier()
        for k in range(K):
            pltpu.async_copy(hp.at[src_v.at[k]], bufs.at[k], gsem[k])

        def rnd(i, _):
            for k in range(K):
                g = K * i + k
                pltpu.make_async_copy(hp.at[src_v.at[g]], bufs.at[k], gsem[k]).wait()
                pltpu.async_copy(bufs.at[k], acc.at[dst_v.at[g]], ssem[k], add=True)
            for k in range(K):
                g = K * i + k
                pltpu.make_async_copy(bufs.at[k], acc.at[dst_v.at[g]], ssem[k]).wait()
                pltpu.async_copy(hp.at[src_v.at[g + K]], bufs.at[k], gsem[k])
            return 0

        lax.fori_loop(0, R - 1, rnd, 0)
        g0 = K * (R - 1)
        for k in range(K):
            pltpu.make_async_copy(hp.at[src_v.at[g0 + k]], bufs.at[k], gsem[k]).wait()
            pltpu.async_copy(bufs.at[k], acc.at[dst_v.at[g0 + k]], ssem[k], add=True)
        for k in range(K):
            pltpu.make_async_copy(bufs.at[k], acc.at[dst_v.at[g0 + k]], ssem[k]).wait()
        plsc.subcore_barrier()
        pltpu.sync_copy(acc.at[pl.ds(s * RPT, RPT)],
                        out.at[c].at[pl.ds(s * RPT, RPT)])
        if not half:
            pltpu.async_copy(zeros.at[pl.ds(s * RPT, RPT)],
                             acc.at[pl.ds(s * RPT, RPT)], ssem[0])


PG = (2 * P) // NW // CH


@functools.partial(
    pl.kernel,
    out_type=jax.ShapeDtypeStruct((2 * P, H), jnp.float32),
    mesh=_mesh,
    scratch_types=[
        pltpu.VMEM((PG, CH), jnp.int32),
        pltpu.VMEM((CH, H), jnp.float32),
        pltpu.VMEM((CH, H), jnp.float32),
        pltpu.SemaphoreType.DMA,
        pltpu.SemaphoreType.DMA,
    ],
)
def _pairs(h3, idxw, out, idx_v, buf_a, buf_b, sem_a, sem_b):
    w = _worker_id()
    base = w * (PG * CH)
    pltpu.sync_copy(idxw.at[w], idx_v)
    bufs = (buf_a, buf_b)
    sems = (sem_a, sem_b)
    pltpu.async_copy(h3.at[idx_v.at[0]], buf_a, sem_a)
    pltpu.async_copy(h3.at[idx_v.at[1]], buf_b, sem_b)
    for g in range(PG):
        b, sm = bufs[g % 2], sems[g % 2]
        pltpu.make_async_copy(h3.at[idx_v.at[g]], b, sm).wait()
        pltpu.sync_copy(b, out.at[pl.ds(base + g * CH, CH)])
        if g + 2 < PG:
            pltpu.async_copy(h3.at[idx_v.at[g + 2]], b, sm)


BR = 2000
GRID_N = N // BR


def _mm_body(x_ref, w_ref, hw_ref):
    hw_ref[...] = jnp.dot(x_ref[...], w_ref[...], precision=_HIGH,
                          preferred_element_type=jnp.float32)


def _sp_body(dp_ref, hw_ref, dis_ref, dinv_ref, hplo_ref, hphi_ref):
    deg = dp_ref[0, :, 0:1] + dp_ref[1, :, 0:1] + 1.0
    dis = lax.rsqrt(deg)
    dis_ref[...] = dis
    dinv_ref[...] = 1.0 / deg
    hp = hw_ref[...] * dis
    hplo_ref[...] = hp[:, :HH]
    hphi_ref[...] = hp[:, HH:]


def _mid_body(slo_ref, shi_ref, hw_ref, dis_ref, dinv_ref, b_ref, w_ref,
              hwo_ref, hplo_ref, hphi_ref):
    dis = dis_ref[...]
    ss = jnp.concatenate([slo_ref[0] + slo_ref[1], shi_ref[0] + shi_ref[1]],
                         axis=1)
    t = dis * ss + dinv_ref[...] * hw_ref[...] + b_ref[...]
    t = jnp.maximum(t, 0.0)
    hw2 = jnp.dot(t, w_ref[...], precision=_HIGH,
                  preferred_element_type=jnp.float32)
    hwo_ref[...] = hw2
    hp = hw2 * dis
    hplo_ref[...] = hp[:, :HH]
    hphi_ref[...] = hp[:, HH:]


def _fin_body(slo_ref, shi_ref, hw_ref, dis_ref, dinv_ref, b_ref, h3_ref):
    ss = jnp.concatenate([slo_ref[0] + slo_ref[1], shi_ref[0] + shi_ref[1]],
                         axis=1)
    h3_ref[...] = (dis_ref[...] * ss + dinv_ref[...] * hw_ref[...] + b_ref[...])


BP = 512
GRID_P = P // BP


def _pred_body(gs_ref, gd_ref, wa_ref, wb_ref, bp1_ref, wp2_ref, bp2_ref, out_ref):
    e = (jnp.dot(gs_ref[...], wa_ref[...], precision=_HIGH,
                 preferred_element_type=jnp.float32)
         + jnp.dot(gd_ref[...], wb_ref[...], precision=_HIGH,
                   preferred_element_type=jnp.float32)
         + bp1_ref[...])
    e = jnp.maximum(e, 0.0)
    z = jnp.sum(e * wp2_ref[...], axis=1, keepdims=True) + bp2_ref[...]
    out_ref[...] = 1.0 / (1.0 + jnp.exp(-z))


def _row_spec(shape):
    return pl.BlockSpec(shape, lambda i: (i, 0))


def _full_spec(shape):
    return pl.BlockSpec(shape, lambda i: (0, 0))


_sh_spec = pl.BlockSpec((NC, BR, HH), lambda i: (0, i, 0))
_d16_spec = pl.BlockSpec((NC, BR, 16), lambda i: (0, i, 0))

_tc_mm = pl.pallas_call(
    _mm_body,
    grid=(GRID_N,),
    in_specs=[_row_spec((BR, H)), _full_spec((H, H))],
    out_specs=_row_spec((BR, H)),
    out_shape=jax.ShapeDtypeStruct((N, H), jnp.float32),
)

_tc_sp = pl.pallas_call(
    _sp_body,
    grid=(GRID_N,),
    in_specs=[_d16_spec, _row_spec((BR, H))],
    out_specs=[_row_spec((BR, 1)), _row_spec((BR, 1)),
               _row_spec((BR, HH)), _row_spec((BR, HH))],
    out_shape=[jax.ShapeDtypeStruct((N, 1), jnp.float32),
               jax.ShapeDtypeStruct((N, 1), jnp.float32),
               jax.ShapeDtypeStruct((N, HH), jnp.float32),
               jax.ShapeDtypeStruct((N, HH), jnp.float32)],
)

_tc_mid = pl.pallas_call(
    _mid_body,
    grid=(GRID_N,),
    in_specs=[_sh_spec, _sh_spec, _row_spec((BR, H)), _row_spec((BR, 1)),
              _row_spec((BR, 1)), _full_spec((1, H)), _full_spec((H, H))],
    out_specs=[_row_spec((BR, H)), _row_spec((BR, HH)), _row_spec((BR, HH))],
    out_shape=[jax.ShapeDtypeStruct((N, H), jnp.float32),
               jax.ShapeDtypeStruct((N, HH), jnp.float32),
               jax.ShapeDtypeStruct((N, HH), jnp.float32)],
)

_tc_fin = pl.pallas_call(
    _fin_body,
    grid=(GRID_N,),
    in_specs=[_sh_spec, _sh_spec, _row_spec((BR, H)), _row_spec((BR, 1)),
              _row_spec((BR, 1)), _full_spec((1, H))],
    out_specs=_row_spec((BR, H)),
    out_shape=jax.ShapeDtypeStruct((N, H), jnp.float32),
)

_tc_pred = pl.pallas_call(
    _pred_body,
    grid=(GRID_P,),
    in_specs=[_row_spec((BP, H)), _row_spec((BP, H)), _full_spec((H, H)),
              _full_spec((H, H)), _full_spec((1, H)), _full_spec((1, H)),
              _full_spec((1, 1))],
    out_specs=_row_spec((BP, 1)),
    out_shape=jax.ShapeDtypeStruct((P, 1), jnp.float32),
)


def kernel(x, edge_index, edge_pairs, W1, b1, W2, b2, W3, b3, Wp1, bp1, Wp2, bp2):
    src = edge_index[0]
    dst = edge_index[1]
    npad = E_PAD - E
    pad_i = jnp.arange(npad, dtype=jnp.int32)
    srcw = jnp.concatenate([src, (pad_i * 997) % N]).reshape(NW, G, CH)
    dstw = jnp.concatenate([dst, N + (pad_i % 16)]).reshape(NW, G, CH)
    idxw = jnp.concatenate([edge_pairs[0], edge_pairs[1]]).reshape(NW, PG, CH)

    zeros_h = jnp.zeros((ACC_ROWS, HH), jnp.float32)
    zeros16 = jnp.zeros((ACC_ROWS, 16), jnp.float32)
    ones16 = jnp.ones((CH, 16), jnp.float32)

    deg_parts = _deg(dstw, ones16, zeros16)
    hw1 = _tc_mm(x, W1)
    dis, dinv, hp1l, hp1h = _tc_sp(deg_parts, hw1)

    s1l, s1h = _agg(hp1l, hp1h, srcw, dstw, zeros_h)
    hw2, hp2l, hp2h = _tc_mid(s1l, s1h, hw1, dis, dinv, b1.reshape(1, H), W2)
    s2l, s2h = _agg(hp2l, hp2h, srcw, dstw, zeros_h)
    hw3, hp3l, hp3h = _tc_mid(s2l, s2h, hw2, dis, dinv, b2.reshape(1, H), W3)
    s3l, s3h = _agg(hp3l, hp3h, srcw, dstw, zeros_h)
    h3 = _tc_fin(s3l, s3h, hw3, dis, dinv, b3.reshape(1, H))

    g = _pairs(h3, idxw)
    pred = _tc_pred(g[:P], g[P:], Wp1[:H], Wp1[H:], bp1.reshape(1, H),
                    Wp2.reshape(1, H), bp2.reshape(1, 1))
    return pred.reshape(P)


# speedup vs baseline: 1.2212x; 1.0666x over previous
import functools

import jax
import jax.numpy as jnp
from jax import lax
from jax.experimental import pallas as pl
from jax.experimental.pallas import tpu as pltpu
from jax.experimental.pallas import tpu_sc as plsc

N = 10000
E = 320000
H = 128
P = 16384

NC = 2
NS = 16
NW = NC * NS
CH = 128
G = 80
EW = G * CH
E_PAD = NW * EW
ACC_ROWS = 10112
RPT = ACC_ROWS // NS

_mesh = plsc.VectorSubcoreMesh(core_axis_name="c", subcore_axis_name="s")
_HIGH = jax.lax.Precision.HIGHEST
_SC_PARAMS = pltpu.CompilerParams(use_tc_tiling_on_sc=False)
HH = H // 2


def _worker_id():
    return lax.axis_index("s") * NC + lax.axis_index("c")


@functools.partial(
    pl.kernel,
    out_type=jax.ShapeDtypeStruct((NC, ACC_ROWS, 16), jnp.float32),
    mesh=_mesh,
    compiler_params=_SC_PARAMS,
    scratch_types=[
        pltpu.VMEM((G, CH), jnp.int32),
        pltpu.VMEM((CH, 16), jnp.float32),
        pltpu.VMEM_SHARED((ACC_ROWS, 16), jnp.float32),
        pltpu.SemaphoreType.DMA,
        pltpu.SemaphoreType.DMA,
        pltpu.SemaphoreType.DMA,
    ],
)
def _deg(dstw, ones, zeros, out, dst_v, ones_v, acc, sem, p0, p1):
    c = lax.axis_index("c")
    s = lax.axis_index("s")
    w = _worker_id()
    pltpu.async_copy(dstw.at[w], dst_v, p0)
    pltpu.async_copy(ones, ones_v, p1)
    pltpu.async_copy(zeros.at[pl.ds(s * RPT, RPT)], acc.at[pl.ds(s * RPT, RPT)], sem)
    pltpu.make_async_copy(dstw.at[w], dst_v, p0).wait()
    pltpu.make_async_copy(ones, ones_v, p1).wait()
    pltpu.make_async_copy(zeros.at[pl.ds(s * RPT, RPT)],
                          acc.at[pl.ds(s * RPT, RPT)], sem).wait()
    plsc.subcore_barrier()

    def fire(g, _):
        pltpu.async_copy(ones_v, acc.at[dst_v.at[g]], sem, add=True)
        return 0

    def drain(g, _):
        pltpu.make_async_copy(ones_v, acc.at[dst_v.at[g]], sem).wait()
        return 0

    lax.fori_loop(0, G, fire, 0)
    lax.fori_loop(0, G, drain, 0)
    plsc.subcore_barrier()
    pltpu.sync_copy(acc.at[pl.ds(s * RPT, RPT)], out.at[c].at[pl.ds(s * RPT, RPT)])


K = 5
EW2 = 2 * EW
G2 = EW2 // CH


@functools.partial(
    pl.kernel,
    out_type=jax.ShapeDtypeStruct((2, ACC_ROWS, HH), jnp.float32),
    mesh=_mesh,
    compiler_params=_SC_PARAMS,
    scratch_types=[
        pltpu.VMEM((G2, CH), jnp.int32),
        pltpu.VMEM((G2, CH), jnp.int32),
        pltpu.VMEM((K, CH, HH), jnp.float32),
        pltpu.VMEM_SHARED((ACC_ROWS, HH), jnp.float32),
    ]
    + [pltpu.SemaphoreType.DMA] * (2 * K),
)
def _agg(hp_s, srcw, dstw, zeros, out, src_v, dst_v, bufs, acc, *sems):
    c = lax.axis_index("c")
    s = lax.axis_index("s")
    gsem = sems[:K]
    ssem = sems[K:]
    pltpu.async_copy(srcw.at[s], src_v, gsem[0])
    pltpu.async_copy(dstw.at[s], dst_v, gsem[1])
    pltpu.async_copy(zeros.at[pl.ds(s * RPT, RPT)],
                     acc.at[pl.ds(s * RPT, RPT)], ssem[0])
    pltpu.make_async_copy(srcw.at[s], src_v, gsem[0]).wait()
    pltpu.make_async_copy(dstw.at[s], dst_v, gsem[1]).wait()
    pltpu.make_async_copy(zeros.at[pl.ds(s * RPT, RPT)],
                          acc.at[pl.ds(s * RPT, RPT)], ssem[0]).wait()
    R = G2 // K
    hp = hp_s.at[c]

    plsc.subcore_barrier()
    for k in range(K):
        pltpu.async_copy(hp.at[src_v.at[k]], bufs.at[k], gsem[k])

    def rnd(i, _):
        for k in range(K):
            g = K * i + k
            pltpu.make_async_copy(hp.at[src_v.at[g]], bufs.at[k], gsem[k]).wait()
            pltpu.async_copy(bufs.at[k], acc.at[dst_v.at[g]], ssem[k], add=True)
        for k in range(K):
            g = K * i + k
            pltpu.make_async_copy(bufs.at[k], acc.at[dst_v.at[g]], ssem[k]).wait()
            pltpu.async_copy(hp.at[src_v.at[g + K]], bufs.at[k], gsem[k])
        return 0

    lax.fori_loop(0, R - 1, rnd, 0)
    g0 = K * (R - 1)
    for k in range(K):
        pltpu.make_async_copy(hp.at[src_v.at[g0 + k]], bufs.at[k], gsem[k]).wait()
        pltpu.async_copy(bufs.at[k], acc.at[dst_v.at[g0 + k]], ssem[k], add=True)
    for k in range(K):
        pltpu.make_async_copy(bufs.at[k], acc.at[dst_v.at[g0 + k]], ssem[k]).wait()
    plsc.subcore_barrier()
    pltpu.sync_copy(acc.at[pl.ds(s * RPT, RPT)],
                    out.at[c].at[pl.ds(s * RPT, RPT)])


PG = (2 * P) // NW // CH


@functools.partial(
    pl.kernel,
    out_type=jax.ShapeDtypeStruct((2 * P, H), jnp.float32),
    mesh=_mesh,
    scratch_types=[
        pltpu.VMEM((PG, CH), jnp.int32),
        pltpu.VMEM((CH, H), jnp.float32),
        pltpu.VMEM((CH, H), jnp.float32),
        pltpu.SemaphoreType.DMA,
        pltpu.SemaphoreType.DMA,
    ],
)
def _pairs(h3, idxw, out, idx_v, buf_a, buf_b, sem_a, sem_b):
    w = _worker_id()
    base = w * (PG * CH)
    pltpu.sync_copy(idxw.at[w], idx_v)
    bufs = (buf_a, buf_b)
    sems = (sem_a, sem_b)
    pltpu.async_copy(h3.at[idx_v.at[0]], buf_a, sem_a)
    pltpu.async_copy(h3.at[idx_v.at[1]], buf_b, sem_b)
    for g in range(PG):
        b, sm = bufs[g % 2], sems[g % 2]
        pltpu.make_async_copy(h3.at[idx_v.at[g]], b, sm).wait()
        pltpu.sync_copy(b, out.at[pl.ds(base + g * CH, CH)])
        if g + 2 < PG:
            pltpu.async_copy(h3.at[idx_v.at[g + 2]], b, sm)


BR = 2000
GRID_N = N // BR


def _mm_body(x_ref, w_ref, hw_ref):
    hw_ref[...] = jnp.dot(x_ref[...], w_ref[...], precision=_HIGH,
                          preferred_element_type=jnp.float32)


def _sp_body(dp_ref, hw_ref, dis_ref, dinv_ref, hp_ref):
    deg = dp_ref[0, :, 0:1] + dp_ref[1, :, 0:1] + 1.0
    dis = lax.rsqrt(deg)
    dis_ref[...] = dis
    dinv_ref[...] = 1.0 / deg
    hp = hw_ref[...] * dis
    hp_ref[0] = hp[:, :HH]
    hp_ref[1] = hp[:, HH:]


def _mid_body(s_ref, hw_ref, dis_ref, dinv_ref, b_ref, w_ref,
              hwo_ref, hp_ref):
    dis = dis_ref[...]
    ss = jnp.concatenate([s_ref[0], s_ref[1]], axis=1)
    t = dis * ss + dinv_ref[...] * hw_ref[...] + b_ref[...]
    t = jnp.maximum(t, 0.0)
    hw2 = jnp.dot(t, w_ref[...], precision=_HIGH,
                  preferred_element_type=jnp.float32)
    hwo_ref[...] = hw2
    hp = hw2 * dis
    hp_ref[0] = hp[:, :HH]
    hp_ref[1] = hp[:, HH:]


def _fin_body(s_ref, hw_ref, dis_ref, dinv_ref, b_ref, h3_ref):
    ss = jnp.concatenate([s_ref[0], s_ref[1]], axis=1)
    h3_ref[...] = (dis_ref[...] * ss + dinv_ref[...] * hw_ref[...] + b_ref[...])


BP = 512
GRID_P = P // BP


def _pred_body(gs_ref, gd_ref, wa_ref, wb_ref, bp1_ref, wp2_ref, bp2_ref, out_ref):
    e = (jnp.dot(gs_ref[...], wa_ref[...], precision=_HIGH,
                 preferred_element_type=jnp.float32)
         + jnp.dot(gd_ref[...], wb_ref[...], precision=_HIGH,
                   preferred_element_type=jnp.float32)
         + bp1_ref[...])
    e = jnp.maximum(e, 0.0)
    z = jnp.sum(e * wp2_ref[...], axis=1, keepdims=True) + bp2_ref[...]
    out_ref[...] = 1.0 / (1.0 + jnp.exp(-z))


def _row_spec(shape):
    return pl.BlockSpec(shape, lambda i: (i, 0))


def _full_spec(shape):
    return pl.BlockSpec(shape, lambda i: (0, 0))


_sh_spec = pl.BlockSpec((2, BR, HH), lambda i: (0, i, 0))
_d16_spec = pl.BlockSpec((NC, BR, 16), lambda i: (0, i, 0))
_hp_shape = jax.ShapeDtypeStruct((2, N, HH), jnp.float32)

_tc_mm = pl.pallas_call(
    _mm_body,
    grid=(GRID_N,),
    in_specs=[_row_spec((BR, H)), _full_spec((H, H))],
    out_specs=_row_spec((BR, H)),
    out_shape=jax.ShapeDtypeStruct((N, H), jnp.float32),
)

_tc_sp = pl.pallas_call(
    _sp_body,
    grid=(GRID_N,),
    in_specs=[_d16_spec, _row_spec((BR, H))],
    out_specs=[_row_spec((BR, 1)), _row_spec((BR, 1)), _sh_spec],
    out_shape=[jax.ShapeDtypeStruct((N, 1), jnp.float32),
               jax.ShapeDtypeStruct((N, 1), jnp.float32),
               _hp_shape],
)

_tc_mid = pl.pallas_call(
    _mid_body,
    grid=(GRID_N,),
    in_specs=[_sh_spec, _row_spec((BR, H)), _row_spec((BR, 1)),
              _row_spec((BR, 1)), _full_spec((1, H)), _full_spec((H, H))],
    out_specs=[_row_spec((BR, H)), _sh_spec],
    out_shape=[jax.ShapeDtypeStruct((N, H), jnp.float32), _hp_shape],
)

_tc_fin = pl.pallas_call(
    _fin_body,
    grid=(GRID_N,),
    in_specs=[_sh_spec, _row_spec((BR, H)), _row_spec((BR, 1)),
              _row_spec((BR, 1)), _full_spec((1, H))],
    out_specs=_row_spec((BR, H)),
    out_shape=jax.ShapeDtypeStruct((N, H), jnp.float32),
)

_tc_pred = pl.pallas_call(
    _pred_body,
    grid=(GRID_P,),
    in_specs=[_row_spec((BP, H)), _row_spec((BP, H)), _full_spec((H, H)),
              _full_spec((H, H)), _full_spec((1, H)), _full_spec((1, H)),
              _full_spec((1, 1))],
    out_specs=_row_spec((BP, 1)),
    out_shape=jax.ShapeDtypeStruct((P, 1), jnp.float32),
)


def kernel(x, edge_index, edge_pairs, W1, b1, W2, b2, W3, b3, Wp1, bp1, Wp2, bp2):
    src = edge_index[0]
    dst = edge_index[1]
    npad = E_PAD - E
    pad_i = jnp.arange(npad, dtype=jnp.int32)
    src_flat = jnp.concatenate([src, (pad_i * 997) % N])
    dst_flat = jnp.concatenate([dst, N + (pad_i % 16)])
    srcw = src_flat.reshape(NS, G2, CH)
    dstw = dst_flat.reshape(NS, G2, CH)
    dstw_deg = dst_flat.reshape(NW, G, CH)
    idxw = jnp.concatenate([edge_pairs[0], edge_pairs[1]]).reshape(NW, PG, CH)

    zeros_h = jnp.zeros((ACC_ROWS, HH), jnp.float32)
    zeros16 = jnp.zeros((ACC_ROWS, 16), jnp.float32)
    ones16 = jnp.ones((CH, 16), jnp.float32)

    deg_parts = _deg(dstw_deg, ones16, zeros16)
    hw1 = _tc_mm(x, W1)
    dis, dinv, hp1 = _tc_sp(deg_parts, hw1)

    s1 = _agg(hp1, srcw, dstw, zeros_h)
    hw2, hp2 = _tc_mid(s1, hw1, dis, dinv, b1.reshape(1, H), W2)
    s2 = _agg(hp2, srcw, dstw, zeros_h)
    hw3, hp3 = _tc_mid(s2, hw2, dis, dinv, b2.reshape(1, H), W3)
    s3 = _agg(hp3, srcw, dstw, zeros_h)
    h3 = _tc_fin(s3, hw3, dis, dinv, b3.reshape(1, H))

    g = _pairs(h3, idxw)
    pred = _tc_pred(g[:P], g[P:], Wp1[:H], Wp1[H:], bp1.reshape(1, H),
                    Wp2.reshape(1, H), bp2.reshape(1, 1))
    return pred.reshape(P)
